# Initial kernel scaffold; baseline (speedup 1.0000x reference)
import jax
import jax.numpy as jnp
from jax.experimental import pallas as pl


def kernel(x, W, b):
    raise NotImplementedError("write your pallas kernel here")


# speedup vs baseline: 1.0163x; 1.0163x over previous
import functools

import jax
import jax.numpy as jnp
from jax.experimental import pallas as pl

TOP_K = 8
BT = 512


def _gate_kernel(x_ref, wt_ref, b_ref, w_out_ref, i_out_ref):
    logits = jnp.dot(x_ref[...], wt_ref[...], preferred_element_type=jnp.float32)
    logits = logits + b_ref[...]
    n = logits.shape[-1]
    iota = jax.lax.broadcasted_iota(jnp.int32, logits.shape, 1)
    vals = logits
    top_vals = []
    top_idx = []
    for _ in range(TOP_K):
        m = jnp.max(vals, axis=-1, keepdims=True)
        idx = jnp.min(jnp.where(vals == m, iota, n), axis=-1, keepdims=True)
        top_vals.append(m)
        top_idx.append(idx)
        vals = jnp.where(iota == idx, -jnp.inf, vals)
    w = jnp.concatenate(top_vals, axis=-1)
    e = jnp.exp(w - w[:, 0:1])
    w_out_ref[...] = e / jnp.sum(e, axis=-1, keepdims=True)
    i_out_ref[...] = jnp.concatenate(top_idx, axis=-1)


@jax.jit
def kernel(x, W, b):
    tokens, d_model = x.shape
    num_experts = W.shape[0]
    wt = W.T
    b2 = b.reshape(1, num_experts)
    grid = (tokens // BT,)
    weights, indices = pl.pallas_call(
        _gate_kernel,
        grid=grid,
        in_specs=[
            pl.BlockSpec((BT, d_model), lambda i: (i, 0)),
            pl.BlockSpec((d_model, num_experts), lambda i: (0, 0)),
            pl.BlockSpec((1, num_experts), lambda i: (0, 0)),
        ],
        out_specs=[
            pl.BlockSpec((BT, TOP_K), lambda i: (i, 0)),
            pl.BlockSpec((BT, TOP_K), lambda i: (i, 0)),
        ],
        out_shape=[
            jax.ShapeDtypeStruct((tokens, TOP_K), jnp.float32),
            jax.ShapeDtypeStruct((tokens, TOP_K), jnp.int32),
        ],
    )(x, wt, b2)
    return (weights, indices)


# speedup vs baseline: 1.1755x; 1.1566x over previous
import functools

import jax
import jax.numpy as jnp
from jax.experimental import pallas as pl

TOP_K = 8
BT = 512


def _gate_kernel(x_ref, wt_ref, b_ref, w_out_ref, i_out_ref):
    logits = jnp.dot(x_ref[...], wt_ref[...], preferred_element_type=jnp.float32)
    logits = logits + b_ref[...]
    n = logits.shape[-1]
    iota = jax.lax.broadcasted_iota(jnp.int32, logits.shape, 1)
    u = jax.lax.bitcast_convert_type(logits, jnp.int32)
    key = jnp.where(u < 0, u ^ jnp.int32(0x7FFFFFFF), u)
    key = (key & jnp.int32(~63)) | (jnp.int32(n - 1) - iota)
    tops = []
    for _ in range(TOP_K):
        m = jnp.max(key, axis=-1, keepdims=True)
        tops.append(m)
        key = jnp.where(key == m, jnp.int32(-0x80000000), key)
    t = jnp.concatenate(tops, axis=-1)
    i_out_ref[...] = jnp.int32(n - 1) - (t & jnp.int32(63))
    vb = t & jnp.int32(~63)
    vb = jnp.where(vb < 0, vb ^ jnp.int32(0x7FFFFFFF), vb)
    w = jax.lax.bitcast_convert_type(vb, jnp.float32)
    e = jnp.exp(w - w[:, 0:1])
    w_out_ref[...] = e / jnp.sum(e, axis=-1, keepdims=True)


@jax.jit
def kernel(x, W, b):
    tokens, d_model = x.shape
    num_experts = W.shape[0]
    wt = W.T
    b2 = b.reshape(1, num_experts)
    grid = (tokens // BT,)
    weights, indices = pl.pallas_call(
        _gate_kernel,
        grid=grid,
        in_specs=[
            pl.BlockSpec((BT, d_model), lambda i: (i, 0)),
            pl.BlockSpec((d_model, num_experts), lambda i: (0, 0)),
            pl.BlockSpec((1, num_experts), lambda i: (0, 0)),
        ],
        out_specs=[
            pl.BlockSpec((BT, TOP_K), lambda i: (i, 0)),
            pl.BlockSpec((BT, TOP_K), lambda i: (i, 0)),
        ],
        out_shape=[
            jax.ShapeDtypeStruct((tokens, TOP_K), jnp.float32),
            jax.ShapeDtypeStruct((tokens, TOP_K), jnp.int32),
        ],
    )(x, wt, b2)
    return (weights, indices)


# speedup vs baseline: 1.2817x; 1.0904x over previous
import functools

import jax
import jax.numpy as jnp
from jax.experimental import pallas as pl

TOP_K = 8
BT = 512


def _gate_kernel(x_ref, wt_ref, b_ref, w_out_ref, i_out_ref):
    logits = jnp.dot(x_ref[...], wt_ref[...], preferred_element_type=jnp.float32)
    logits = logits + b_ref[...]
    n = logits.shape[-1]
    iota = jax.lax.broadcasted_iota(jnp.int32, logits.shape, 1)
    u = jax.lax.bitcast_convert_type(logits, jnp.int32)
    key = jnp.where(u < 0, u ^ jnp.int32(0x7FFFFFFF), u)
    key = (key & jnp.int32(~63)) | (jnp.int32(n - 1) - iota)
    keyf = jax.lax.bitcast_convert_type(
        jnp.where(key < 0, key ^ jnp.int32(0x7FFFFFFF), key), jnp.float32)
    tops = []
    for _ in range(TOP_K):
        m = jnp.max(keyf, axis=-1, keepdims=True)
        tops.append(m)
        keyf = jnp.where(keyf == m, -jnp.inf, keyf)
    tb = jax.lax.bitcast_convert_type(jnp.concatenate(tops, axis=-1), jnp.int32)
    t = jnp.where(tb < 0, tb ^ jnp.int32(0x7FFFFFFF), tb)
    i_out_ref[...] = jnp.int32(n - 1) - (t & jnp.int32(63))
    vb = t & jnp.int32(~63)
    vb = jnp.where(vb < 0, vb ^ jnp.int32(0x7FFFFFFF), vb)
    w = jax.lax.bitcast_convert_type(vb, jnp.float32)
    e = jnp.exp(w - w[:, 0:1])
    w_out_ref[...] = e / jnp.sum(e, axis=-1, keepdims=True)


@jax.jit
def kernel(x, W, b):
    tokens, d_model = x.shape
    num_experts = W.shape[0]
    wt = W.T
    b2 = b.reshape(1, num_experts)
    grid = (tokens // BT,)
    weights, indices = pl.pallas_call(
        _gate_kernel,
        grid=grid,
        in_specs=[
            pl.BlockSpec((BT, d_model), lambda i: (i, 0)),
            pl.BlockSpec((d_model, num_experts), lambda i: (0, 0)),
            pl.BlockSpec((1, num_experts), lambda i: (0, 0)),
        ],
        out_specs=[
            pl.BlockSpec((BT, TOP_K), lambda i: (i, 0)),
            pl.BlockSpec((BT, TOP_K), lambda i: (i, 0)),
        ],
        out_shape=[
            jax.ShapeDtypeStruct((tokens, TOP_K), jnp.float32),
            jax.ShapeDtypeStruct((tokens, TOP_K), jnp.int32),
        ],
    )(x, wt, b2)
    return (weights, indices)


# speedup vs baseline: 1.3993x; 1.0917x over previous
import functools

import jax
import jax.numpy as jnp
from jax.experimental import pallas as pl

TOP_K = 8
BT = 1024


def _gate_kernel(x_ref, wt_ref, b_ref, w_out_ref, i_out_ref):
    logits = jnp.dot(x_ref[...], wt_ref[...], preferred_element_type=jnp.float32)
    logits = logits + b_ref[...]
    n = logits.shape[-1]
    iota = jax.lax.broadcasted_iota(jnp.int32, logits.shape, 1)
    u = jax.lax.bitcast_convert_type(logits, jnp.int32)
    key = jnp.where(u < 0, u ^ jnp.int32(0x7FFFFFFF), u)
    key = (key & jnp.int32(~63)) | (jnp.int32(n - 1) - iota)
    keyf = jax.lax.bitcast_convert_type(
        jnp.where(key < 0, key ^ jnp.int32(0x7FFFFFFF), key), jnp.float32)
    tops = []
    for _ in range(TOP_K):
        m = jnp.max(keyf, axis=-1, keepdims=True)
        tops.append(m)
        keyf = jnp.where(keyf == m, -jnp.inf, keyf)
    tb = jax.lax.bitcast_convert_type(jnp.concatenate(tops, axis=-1), jnp.int32)
    t = jnp.where(tb < 0, tb ^ jnp.int32(0x7FFFFFFF), tb)
    i_out_ref[...] = jnp.int32(n - 1) - (t & jnp.int32(63))
    vb = t & jnp.int32(~63)
    vb = jnp.where(vb < 0, vb ^ jnp.int32(0x7FFFFFFF), vb)
    w = jax.lax.bitcast_convert_type(vb, jnp.float32)
    e = jnp.exp(w - w[:, 0:1])
    w_out_ref[...] = e / jnp.sum(e, axis=-1, keepdims=True)


@jax.jit
def kernel(x, W, b):
    tokens, d_model = x.shape
    num_experts = W.shape[0]
    wt = W.T
    b2 = b.reshape(1, num_experts)
    grid = (tokens // BT,)
    weights, indices = pl.pallas_call(
        _gate_kernel,
        grid=grid,
        in_specs=[
            pl.BlockSpec((BT, d_model), lambda i: (i, 0)),
            pl.BlockSpec((d_model, num_experts), lambda i: (0, 0)),
            pl.BlockSpec((1, num_experts), lambda i: (0, 0)),
        ],
        out_specs=[
            pl.BlockSpec((BT, TOP_K), lambda i: (i, 0)),
            pl.BlockSpec((BT, TOP_K), lambda i: (i, 0)),
        ],
        out_shape=[
            jax.ShapeDtypeStruct((tokens, TOP_K), jnp.float32),
            jax.ShapeDtypeStruct((tokens, TOP_K), jnp.int32),
        ],
    )(x, wt, b2)
    return (weights, indices)
